# baseline (speedup 1.0000x reference)
import functools

import jax
import jax.numpy as jnp
from jax import lax
from jax.experimental import pallas as pl
from jax.experimental.pallas import tpu as pltpu
from jax.experimental.pallas import tpu_sc as plsc

_NC = 2
_NS = 16
_NW = _NC * _NS
_L = 16


def _anchor_gather(vertices, idx, B, N, K, C):
    E = N * C
    E8 = E // 8
    V8 = B * E8
    BPW = B // _NW
    KP = 48
    RPB = 2 * KP
    RS = 16
    NT = (K * C + _L - 1) // _L + 1
    PW = NT * _L
    OW = BPW * PW

    table = vertices.reshape(V8, 8)
    mesh = plsc.VectorSubcoreMesh(core_axis_name="c", subcore_axis_name="s",
                                  num_cores=_NC, num_subcores=_NS)

    @functools.partial(
        pl.kernel,
        out_type=jax.ShapeDtypeStruct((B * PW,), jnp.float32),
        mesh=mesh,
        scratch_types=[
            pltpu.VMEM((K,), jnp.int32),
            pltpu.VMEM((RS, RPB), jnp.int32),
            pltpu.VMEM((RS, RPB, 8), jnp.float32),
            pltpu.VMEM((OW,), jnp.float32),
            pltpu.SemaphoreType.DMA,
        ],
        compiler_params=pltpu.CompilerParams(use_tc_tiling_on_sc=False,
                                             needs_layout_passes=False),
    )
    def k(table_hbm, idx_hbm, out_hbm, idxv, bidx, staged, outv, sem):
        wid = lax.axis_index("s") * _NC + lax.axis_index("c")
        base_b = wid * BPW
        pltpu.sync_copy(idx_hbm, idxv)

        io = lax.iota(jnp.int32, _L)

        uvecs = []
        for t in range(KP // _L):
            kv = jnp.minimum(io + t * _L, K - 1)
            uvecs.append((plsc.load_gather(idxv, [kv]) * 3) >> 3)

        kks, wos = [], []
        for t in range(NT):
            j = io + t * _L
            pos = j // 3
            fv = (plsc.load_gather(idxv, [jnp.minimum(pos, K - 1)]) * 3) & 7
            tt = fv + (j - pos * 3)
            kks.append(pos + KP * (tt >> 3))
            wos.append(tt & 7)

        G = 8

        def group(g, carry):
            b0 = g * G
            for i in range(G):
                b = b0 + i
                s = b % RS
                w0 = (base_b + b) * E8
                for t in range(KP // _L):
                    lo = uvecs[t] + w0
                    bidx[s, pl.ds(t * _L, _L)] = lo
                    bidx[s, pl.ds(KP + t * _L, _L)] = jnp.minimum(lo + 1,
                                                                  V8 - 1)
                pltpu.async_copy(table_hbm.at[bidx.at[s]], staged.at[s], sem)
            for i in range(G):
                s = (b0 + i) % RS
                pltpu.make_async_copy(table_hbm.at[bidx.at[s]],
                                      staged.at[s], sem).wait()
            for i in range(G):
                b = b0 + i
                sv = jnp.full((_L,), b % RS, dtype=jnp.int32)
                for t in range(NT):
                    outv[pl.ds(b * PW + t * _L, _L)] = plsc.load_gather(
                        staged, [sv, kks[t], wos[t]])
            return carry

        lax.fori_loop(0, BPW // G, group, 0)
        pltpu.sync_copy(outv, out_hbm.at[pl.ds(wid * OW, OW)])

    return k(table, idx)


def kernel(vertices, vert_idx):
    B, N, C = vertices.shape
    (K,) = vert_idx.shape
    idx = vert_idx.astype(jnp.int32)
    out = _anchor_gather(vertices, idx, B, N, K, C)
    pw = ((K * C + 15) // 16 + 1) * 16
    return out.reshape(B, pw)[:, :K * C].reshape(B, K, C)
